# Initial kernel scaffold; baseline (speedup 1.0000x reference)
import jax
import jax.numpy as jnp
from jax.experimental import pallas as pl


def kernel(inputs, table, W1, b1, W2, b2):
    raise NotImplementedError("write your pallas kernel here")


# speedup vs baseline: 17.5531x; 17.5531x over previous
import functools

import jax
import jax.numpy as jnp
from jax import lax
from jax.experimental import pallas as pl
from jax.experimental.pallas import tpu as pltpu
from jax.experimental.pallas import tpu_sc as plsc

B = 4096
L = 200
D = 64
H = 64
NCLS = 2

_L0 = 104
_L1 = L - _L0
NBUF = 4
LOOKAHEAD = 3
UNROLL = 8
_N_ITERS = L // UNROLL


def _sc_pool(idx, table):
    info = plsc.get_sparse_core_info()
    nc, ns = info.num_cores, info.num_subcores
    nw = nc * ns
    bpw = B // nw

    mesh = plsc.VectorSubcoreMesh(core_axis_name="c", subcore_axis_name="s")

    @functools.partial(
        pl.kernel,
        mesh=mesh,
        compiler_params=pltpu.CompilerParams(use_tc_tiling_on_sc=False),
        out_type=jax.ShapeDtypeStruct((B, D), jnp.float32),
        scratch_types=[
            pltpu.VMEM((bpw, L), jnp.int32),
            pltpu.VMEM((NBUF, L, D), jnp.float32),
            pltpu.VMEM((bpw, D), jnp.float32),
            pltpu.SemaphoreType.DMA,
            pltpu.SemaphoreType.DMA,
            pltpu.SemaphoreType.DMA,
            pltpu.SemaphoreType.DMA,
        ],
    )
    def k(idx_hbm, table_hbm, out_hbm, idx_v, rows_v, pooled_v, s0, s1, s2, s3):
        sems = (s0, s1, s2, s3)
        wid = lax.axis_index("s") * nc + lax.axis_index("c")
        base = wid * bpw
        pltpu.sync_copy(idx_hbm.at[pl.ds(base, bpw)], idx_v)

        def issue(b, u):
            pltpu.make_async_copy(
                table_hbm.at[idx_v.at[b, pl.ds(0, _L0)]],
                rows_v.at[u, pl.ds(0, _L0)],
                sems[u],
            ).start()
            pltpu.make_async_copy(
                table_hbm.at[idx_v.at[b, pl.ds(_L0, _L1)]],
                rows_v.at[u, pl.ds(_L0, _L1)],
                sems[u],
            ).start()

        def wait(u):
            pltpu.make_async_copy(
                table_hbm.at[idx_v.at[0, pl.ds(0, _L0)]],
                rows_v.at[u, pl.ds(0, _L0)],
                sems[u],
            ).wait()
            pltpu.make_async_copy(
                table_hbm.at[idx_v.at[0, pl.ds(_L0, _L1)]],
                rows_v.at[u, pl.ds(_L0, _L1)],
                sems[u],
            ).wait()

        for u in range(LOOKAHEAD):
            issue(u, u)

        scale = jnp.float32(1.0 / L)

        def group(g, carry):
            for u in range(NBUF):
                b = g * NBUF + u
                wait(u)
                nb = b + LOOKAHEAD

                @pl.when(nb < bpw)
                def _():
                    issue(nb, (u + LOOKAHEAD) % NBUF)

                def acc_body(i, acc):
                    a0, a1, a2, a3 = acc
                    r0 = i * UNROLL
                    for r in range(UNROLL):
                        row = r0 + r
                        a0 = a0 + rows_v[u, row, pl.ds(0, 16)]
                        a1 = a1 + rows_v[u, row, pl.ds(16, 16)]
                        a2 = a2 + rows_v[u, row, pl.ds(32, 16)]
                        a3 = a3 + rows_v[u, row, pl.ds(48, 16)]
                    return a0, a1, a2, a3

                z = jnp.zeros((16,), jnp.float32)
                a0, a1, a2, a3 = lax.fori_loop(0, _N_ITERS, acc_body, (z, z, z, z))
                pooled_v[b, pl.ds(0, 16)] = a0 * scale
                pooled_v[b, pl.ds(16, 16)] = a1 * scale
                pooled_v[b, pl.ds(32, 16)] = a2 * scale
                pooled_v[b, pl.ds(48, 16)] = a3 * scale
            return carry

        lax.fori_loop(0, bpw // NBUF, group, 0)
        pltpu.sync_copy(pooled_v, out_hbm.at[pl.ds(base, bpw)])

    return k(idx, table)


def _mlp_body(x_ref, w1_ref, b1_ref, w2_ref, b2_ref, o_ref):
    h = jnp.dot(x_ref[...], w1_ref[...], preferred_element_type=jnp.float32)
    h = jnp.maximum(h + b1_ref[...], 0.0)
    o_ref[...] = jnp.dot(h, w2_ref[...], preferred_element_type=jnp.float32) + b2_ref[...]


def _tc_mlp(pooled, W1, b1, W2p, b2p):
    return pl.pallas_call(
        _mlp_body,
        out_shape=jax.ShapeDtypeStruct((B, 128), jnp.float32),
    )(pooled, W1, b1, W2p, b2p)


def kernel(inputs, table, W1, b1, W2, b2):
    idx = inputs.astype(jnp.int32)
    pooled = _sc_pool(idx, table)
    W2p = jnp.pad(W2, ((0, 0), (0, 128 - NCLS)))
    b2p = jnp.pad(b2, (0, 128 - NCLS)).reshape(1, 128)
    logits = _tc_mlp(pooled, W1, b1.reshape(1, H), W2p, b2p)
    return logits[:, :NCLS]


# speedup vs baseline: 18.2810x; 1.0415x over previous
import functools

import jax
import jax.numpy as jnp
from jax import lax
from jax.experimental import pallas as pl
from jax.experimental.pallas import tpu as pltpu
from jax.experimental.pallas import tpu_sc as plsc

B = 4096
L = 200
D = 64
H = 64
NCLS = 2

_L0 = 104
_L1 = L - _L0
NBUF = 4
LOOKAHEAD = 3
UNROLL = 8
_N_ITERS = L // UNROLL


def _sc_pool(idx, table):
    info = plsc.get_sparse_core_info()
    nc, ns = info.num_cores, info.num_subcores
    nw = nc * ns
    bpw = B // nw

    mesh = plsc.VectorSubcoreMesh(core_axis_name="c", subcore_axis_name="s")

    @functools.partial(
        pl.kernel,
        mesh=mesh,
        compiler_params=pltpu.CompilerParams(use_tc_tiling_on_sc=False),
        out_type=jax.ShapeDtypeStruct((B, D), jnp.float32),
        scratch_types=[
            pltpu.VMEM((bpw, L), jnp.int32),
            pltpu.VMEM((NBUF, L, D), jnp.float32),
            pltpu.VMEM((bpw, D), jnp.float32),
            pltpu.SemaphoreType.DMA,
            pltpu.SemaphoreType.DMA,
            pltpu.SemaphoreType.DMA,
            pltpu.SemaphoreType.DMA,
        ],
    )
    def k(idx_hbm, table_hbm, out_hbm, idx_v, rows_v, pooled_v, s0, s1, s2, s3):
        sems = (s0, s1, s2, s3)
        wid = lax.axis_index("s") * nc + lax.axis_index("c")
        base = wid * bpw
        pltpu.sync_copy(idx_hbm.at[pl.ds(base, bpw)], idx_v)

        def issue(b, u):
            pltpu.make_async_copy(
                table_hbm.at[idx_v.at[b, pl.ds(0, _L0)]],
                rows_v.at[u, pl.ds(0, _L0)],
                sems[u],
            ).start()
            pltpu.make_async_copy(
                table_hbm.at[idx_v.at[b, pl.ds(_L0, _L1)]],
                rows_v.at[u, pl.ds(_L0, _L1)],
                sems[u],
            ).start()

        def wait(u):
            pltpu.make_async_copy(
                table_hbm.at[idx_v.at[0, pl.ds(0, _L0)]],
                rows_v.at[u, pl.ds(0, _L0)],
                sems[u],
            ).wait()
            pltpu.make_async_copy(
                table_hbm.at[idx_v.at[0, pl.ds(_L0, _L1)]],
                rows_v.at[u, pl.ds(_L0, _L1)],
                sems[u],
            ).wait()

        for u in range(LOOKAHEAD):
            issue(u, u)

        def group(g, carry):
            for u in range(NBUF):
                b = g * NBUF + u
                wait(u)
                nb = b + LOOKAHEAD

                @pl.when(nb < bpw)
                def _():
                    issue(nb, (u + LOOKAHEAD) % NBUF)

                def acc_body(i, acc):
                    a0, a1, a2, a3 = acc
                    r0 = i * UNROLL
                    for r in range(UNROLL):
                        row = r0 + r
                        a0 = a0 + rows_v[u, row, pl.ds(0, 16)]
                        a1 = a1 + rows_v[u, row, pl.ds(16, 16)]
                        a2 = a2 + rows_v[u, row, pl.ds(32, 16)]
                        a3 = a3 + rows_v[u, row, pl.ds(48, 16)]
                    return a0, a1, a2, a3

                z = jnp.zeros((16,), jnp.float32)
                a0, a1, a2, a3 = lax.fori_loop(0, _N_ITERS, acc_body, (z, z, z, z))
                pooled_v[b, pl.ds(0, 16)] = a0
                pooled_v[b, pl.ds(16, 16)] = a1
                pooled_v[b, pl.ds(32, 16)] = a2
                pooled_v[b, pl.ds(48, 16)] = a3
            return carry

        lax.fori_loop(0, bpw // NBUF, group, 0)
        pltpu.sync_copy(pooled_v, out_hbm.at[pl.ds(base, bpw)])

    return k(idx, table)


def _mlp_body(x_ref, w1_ref, b1_ref, w2_ref, b2_ref, o_ref):
    h = jnp.dot(x_ref[...], w1_ref[...], preferred_element_type=jnp.float32)
    h = jnp.maximum(h + b1_ref[...], 0.0)
    o_ref[...] = jnp.dot(h, w2_ref[...], preferred_element_type=jnp.float32) + b2_ref[...]


def _tc_mlp(pooled, W1, b1, W2p, b2p):
    return pl.pallas_call(
        _mlp_body,
        out_shape=jax.ShapeDtypeStruct((B, 128), jnp.float32),
    )(pooled, W1, b1, W2p, b2p)


def kernel(inputs, table, W1, b1, W2, b2):
    idx = inputs.astype(jnp.int32) * 2
    table2 = jnp.pad(table, ((0, 0), (0, D))).reshape(2 * table.shape[0], D)
    pooled = _sc_pool(idx, table2)
    W1s = W1 * jnp.float32(1.0 / L)
    W2p = jnp.pad(W2, ((0, 0), (0, 128 - NCLS)))
    b2p = jnp.pad(b2, (0, 128 - NCLS)).reshape(1, 128)
    logits = _tc_mlp(pooled, W1s, b1.reshape(1, H), W2p, b2p)
    return logits[:, :NCLS]
